# baseline (speedup 1.0000x reference)
import jax
import jax.numpy as jnp
import numpy as np
from jax.experimental import pallas as pl


def _dg(a, b, ca, cb):
    return jax.lax.dot_general(
        a, b, (((ca,), (cb,)), ((), ())), preferred_element_type=jnp.float32
    )


def _cheb_terms(A, dinv, x):
    t1 = -(dinv * _dg(A, dinv * x, 1, 0))
    t2 = 2.0 * (-(dinv * _dg(A, dinv * t1, 1, 0))) - x
    return t1, t2


def _dinv_of(A):
    deg = jnp.sum(A, axis=1, keepdims=True)
    pos = deg > 0
    return jnp.where(pos, 1.0 / jnp.sqrt(jnp.where(pos, deg, 1.0)), 0.0)


def _adj_body(x1_r, x2_r, x3_r, Wm1_r, bm1_r, Wm2_r, bm2_r,
              feats_o, tx1f_o, tx2f_o, tx1m_o, tx2m_o, ac_o, dinvc_o):
    x1 = x1_r[:]
    x2 = x2_r[:]
    n = x1.shape[0]
    fusion = jnp.concatenate([x1, x2], axis=1)
    nrm = jnp.maximum(
        jnp.sqrt(jnp.sum(fusion * fusion, axis=1, keepdims=True)), 1e-12
    )
    xn = (fusion / nrm).astype(jnp.bfloat16)
    sim = _dg(xn, xn, 1, 1)

    x3 = x3_r[:]
    pd = x3.shape[1]
    mu = jnp.mean(x3, axis=0, keepdims=True)
    sd = jnp.sqrt(jnp.mean((x3 - mu) * (x3 - mu), axis=0, keepdims=True))
    x3s = ((x3 - mu) / sd).astype(jnp.bfloat16)
    Wm1 = Wm1_r[:].astype(jnp.bfloat16)
    a = _dg(x3s, Wm1[:pd, :], 1, 0)
    bT = _dg(Wm1[pd:, :], x3s, 0, 1)
    bm1 = bm1_r[:]
    Wm2 = Wm2_r[:].astype(jnp.bfloat16).astype(jnp.float32)
    s = None
    for h in range(pd):
        t = jax.nn.relu(a[:, h:h + 1] + bT[h:h + 1, :] + bm1[0:1, h:h + 1])
        t = t.astype(jnp.bfloat16).astype(jnp.float32) * Wm2[h:h + 1, 0:1]
        s = t if s is None else s + t
    meta = jax.nn.sigmoid(s + bm2_r[0:1, 0:1])

    ii = jax.lax.broadcasted_iota(jnp.int32, (n, n), 0)
    jj = jax.lax.broadcasted_iota(jnp.int32, (n, n), 1)
    offd = ii != jj
    Af = jnp.where(offd & (sim > 0.8), sim, 0.0)
    Am = jnp.where(offd & (meta > 0.8), meta, 0.0)
    comb = sim + meta
    Ac = jnp.where(offd & (comb > 1.6), comb, 0.0)

    dinv_f = _dinv_of(Af)
    t1f, t2f = _cheb_terms(Af, dinv_f, fusion)
    tx1f_o[:] = t1f
    tx2f_o[:] = t2f

    dinv_m = _dinv_of(Am)
    t1m, t2m = _cheb_terms(Am, dinv_m, fusion)
    tx1m_o[:] = t1m
    tx2m_o[:] = t2m

    feats_o[:] = fusion
    ac_o[:] = Ac
    dinvc_o[:] = _dinv_of(Ac)


def _main_body(feats_r, tx1f_r, tx2f_r, tx1m_r, tx2m_r, ac_r, dinvc_r,
               Wgc1_r, Wgc2_r, Wsgc_r, Wq_r, bq_r, Wk_r, bk_r, Wv_r, bv_r,
               Wg0_r, Wg1_r, Wg2_r, Wg3_r, Wc1_r, bc1_r, gamma_r, beta_r,
               Wc2_r, bc2_r, out_o):
    f = feats_r[:]
    t1f = tx1f_r[:]
    t2f = tx2f_r[:]
    t1m = tx1m_r[:]
    t2m = tx2m_r[:]
    din = f.shape[1]

    x1o = jax.nn.relu(
        _dg(f, Wgc1_r[0, :, :], 1, 0)
        + _dg(t1f, Wgc1_r[1, :, :], 1, 0)
        + _dg(t2f, Wgc1_r[2, :, :], 1, 0)
    )
    x2o = jax.nn.relu(
        _dg(f, Wgc2_r[0, :, :], 1, 0)
        + _dg(t1m, Wgc2_r[1, :, :], 1, 0)
        + _dg(t2m, Wgc2_r[2, :, :], 1, 0)
    )
    xm = (
        _dg(f, Wsgc_r[0, :, :], 1, 0)
        + _dg(t1f, Wsgc_r[1, :, :], 1, 0)
        + _dg(t2f, Wsgc_r[2, :, :], 1, 0)
    )

    def qkv(W_r, b_r):
        W = W_r[:]
        return (
            _dg(x1o, W[0:din, :], 1, 0)
            + _dg(x2o, W[din:2 * din, :], 1, 0)
            + _dg(xm, W[2 * din:, :], 1, 0)
            + b_r[:]
        )

    Q = qkv(Wq_r, bq_r)
    K = qkv(Wk_r, bk_r)
    V = qkv(Wv_r, bv_r)

    S = _dg(Q, K, 0, 0) * np.float32(1.0 / np.sqrt(float(din)))
    m = jnp.max(S, axis=0, keepdims=True)
    e = jnp.exp(S - m)
    att = e / jnp.sum(e, axis=0, keepdims=True)
    xa = _dg(V, att, 1, 0)

    A = ac_r[:]
    dinv = dinvc_r[:]

    def cheb(x, W_r):
        t1 = -(dinv * _dg(A, dinv * x, 1, 0))
        t2 = 2.0 * (-(dinv * _dg(A, dinv * t1, 1, 0))) - x
        return (
            _dg(x, W_r[0, :, :], 1, 0)
            + _dg(t1, W_r[1, :, :], 1, 0)
            + _dg(t2, W_r[2, :, :], 1, 0)
        )

    h0 = jax.nn.relu(cheb(xa, Wg0_r))
    h1 = jax.nn.relu(cheb(h0, Wg1_r))
    h2 = jax.nn.relu(cheb(h1, Wg2_r))
    h3 = jax.nn.relu(cheb(h2, Wg3_r))

    hg = h0.shape[1]
    Wc1 = Wc1_r[:]
    hc = jax.nn.relu(
        _dg(h0, Wc1[0:hg, :], 1, 0)
        + _dg(h1, Wc1[hg:2 * hg, :], 1, 0)
        + _dg(h2, Wc1[2 * hg:3 * hg, :], 1, 0)
        + _dg(h3, Wc1[3 * hg:, :], 1, 0)
        + bc1_r[:]
    )
    hc = hc / jnp.sqrt(1.0 + 1e-5) * gamma_r[:] + beta_r[:]
    logit = _dg(hc, Wc2_r[:], 1, 0) + bc2_r[:]
    m2 = jnp.max(logit, axis=1, keepdims=True)
    e2 = jnp.exp(logit - m2)
    out_o[:] = e2 / jnp.sum(e2, axis=1, keepdims=True)


def kernel(x1, x2, x3, Wq, bq, Wk, bk, Wv, bv, Wm1, bm1, Wm2, bm2,
           Wgc1, Wgc2, Wsgc, Wg0, Wg1, Wg2, Wg3, Wc1, bc1, gamma, beta,
           Wc2, bc2):
    n = x1.shape[0]
    din = x1.shape[1] + x2.shape[1]
    f32 = jnp.float32
    r2 = lambda v: v.reshape(1, -1)

    feats, tx1f, tx2f, tx1m, tx2m, ac, dinvc = pl.pallas_call(
        _adj_body,
        out_shape=(
            jax.ShapeDtypeStruct((n, din), f32),
            jax.ShapeDtypeStruct((n, din), f32),
            jax.ShapeDtypeStruct((n, din), f32),
            jax.ShapeDtypeStruct((n, din), f32),
            jax.ShapeDtypeStruct((n, din), f32),
            jax.ShapeDtypeStruct((n, n), f32),
            jax.ShapeDtypeStruct((n, 1), f32),
        ),
    )(x1, x2, x3, Wm1, r2(bm1), Wm2, r2(bm2))

    out = pl.pallas_call(
        _main_body,
        out_shape=jax.ShapeDtypeStruct((n, Wc2.shape[1]), f32),
    )(feats, tx1f, tx2f, tx1m, tx2m, ac, dinvc,
      Wgc1, Wgc2, Wsgc, Wq, r2(bq), Wk, r2(bk), Wv, r2(bv),
      Wg0, Wg1, Wg2, Wg3, Wc1, r2(bc1), r2(gamma), r2(beta), Wc2, r2(bc2))
    return out


# speedup vs baseline: 1.0049x; 1.0049x over previous
import jax
import jax.numpy as jnp
import numpy as np
from jax.experimental import pallas as pl


def _dg(a, b, ca, cb):
    return jax.lax.dot_general(
        a.astype(jnp.bfloat16), b.astype(jnp.bfloat16),
        (((ca,), (cb,)), ((), ())), preferred_element_type=jnp.float32
    )


def _cheb_terms(A, dinv, x):
    t1 = -(dinv * _dg(A, dinv * x, 1, 0))
    t2 = 2.0 * (-(dinv * _dg(A, dinv * t1, 1, 0))) - x
    return t1, t2


def _dinv_of(A):
    deg = jnp.sum(A, axis=1, keepdims=True)
    pos = deg > 0
    return jnp.where(pos, 1.0 / jnp.sqrt(jnp.where(pos, deg, 1.0)), 0.0)


def _adj_body(x1_r, x2_r, x3_r, Wm1_r, bm1_r, Wm2_r, bm2_r,
              feats_o, tx1f_o, tx2f_o, tx1m_o, tx2m_o, ac_o, dinvc_o):
    x1 = x1_r[:]
    x2 = x2_r[:]
    n = x1.shape[0]
    fusion = jnp.concatenate([x1, x2], axis=1)
    nrm = jnp.maximum(
        jnp.sqrt(jnp.sum(fusion * fusion, axis=1, keepdims=True)), 1e-12
    )
    xn = (fusion / nrm).astype(jnp.bfloat16)
    sim = _dg(xn, xn, 1, 1)

    x3 = x3_r[:]
    pd = x3.shape[1]
    mu = jnp.mean(x3, axis=0, keepdims=True)
    sd = jnp.sqrt(jnp.mean((x3 - mu) * (x3 - mu), axis=0, keepdims=True))
    x3s = ((x3 - mu) / sd).astype(jnp.bfloat16)
    Wm1 = Wm1_r[:].astype(jnp.bfloat16)
    a = _dg(x3s, Wm1[:pd, :], 1, 0)
    bT = _dg(Wm1[pd:, :], x3s, 0, 1)
    bm1 = bm1_r[:]
    Wm2 = Wm2_r[:].astype(jnp.bfloat16).astype(jnp.float32)
    s = None
    for h in range(pd):
        t = jax.nn.relu(a[:, h:h + 1] + bT[h:h + 1, :] + bm1[0:1, h:h + 1])
        t = t.astype(jnp.bfloat16).astype(jnp.float32) * Wm2[h:h + 1, 0:1]
        s = t if s is None else s + t
    meta = jax.nn.sigmoid(s + bm2_r[0:1, 0:1])

    ii = jax.lax.broadcasted_iota(jnp.int32, (n, n), 0)
    jj = jax.lax.broadcasted_iota(jnp.int32, (n, n), 1)
    offd = ii != jj
    Af = jnp.where(offd & (sim > 0.8), sim, 0.0)
    Am = jnp.where(offd & (meta > 0.8), meta, 0.0)
    comb = sim + meta
    Ac = jnp.where(offd & (comb > 1.6), comb, 0.0)

    dinv_f = _dinv_of(Af)
    t1f, t2f = _cheb_terms(Af, dinv_f, fusion)
    tx1f_o[:] = t1f
    tx2f_o[:] = t2f

    dinv_m = _dinv_of(Am)
    t1m, t2m = _cheb_terms(Am, dinv_m, fusion)
    tx1m_o[:] = t1m
    tx2m_o[:] = t2m

    feats_o[:] = fusion
    ac_o[:] = Ac
    dinvc_o[:] = _dinv_of(Ac)


def _main_body(feats_r, tx1f_r, tx2f_r, tx1m_r, tx2m_r, ac_r, dinvc_r,
               Wgc1_r, Wgc2_r, Wsgc_r, Wq_r, bq_r, Wk_r, bk_r, Wv_r, bv_r,
               Wg0_r, Wg1_r, Wg2_r, Wg3_r, Wc1_r, bc1_r, gamma_r, beta_r,
               Wc2_r, bc2_r, out_o):
    f = feats_r[:]
    t1f = tx1f_r[:]
    t2f = tx2f_r[:]
    t1m = tx1m_r[:]
    t2m = tx2m_r[:]
    din = f.shape[1]

    x1o = jax.nn.relu(
        _dg(f, Wgc1_r[0, :, :], 1, 0)
        + _dg(t1f, Wgc1_r[1, :, :], 1, 0)
        + _dg(t2f, Wgc1_r[2, :, :], 1, 0)
    )
    x2o = jax.nn.relu(
        _dg(f, Wgc2_r[0, :, :], 1, 0)
        + _dg(t1m, Wgc2_r[1, :, :], 1, 0)
        + _dg(t2m, Wgc2_r[2, :, :], 1, 0)
    )
    xm = (
        _dg(f, Wsgc_r[0, :, :], 1, 0)
        + _dg(t1f, Wsgc_r[1, :, :], 1, 0)
        + _dg(t2f, Wsgc_r[2, :, :], 1, 0)
    )

    def qkv(W_r, b_r):
        W = W_r[:]
        return (
            _dg(x1o, W[0:din, :], 1, 0)
            + _dg(x2o, W[din:2 * din, :], 1, 0)
            + _dg(xm, W[2 * din:, :], 1, 0)
            + b_r[:]
        )

    Q = qkv(Wq_r, bq_r)
    K = qkv(Wk_r, bk_r)
    V = qkv(Wv_r, bv_r)

    S = _dg(Q, K, 0, 0) * np.float32(1.0 / np.sqrt(float(din)))
    m = jnp.max(S, axis=0, keepdims=True)
    e = jnp.exp(S - m)
    att = e / jnp.sum(e, axis=0, keepdims=True)
    xa = _dg(V, att, 1, 0)

    A = ac_r[:]
    dinv = dinvc_r[:]

    def cheb(x, W_r):
        t1 = -(dinv * _dg(A, dinv * x, 1, 0))
        t2 = 2.0 * (-(dinv * _dg(A, dinv * t1, 1, 0))) - x
        return (
            _dg(x, W_r[0, :, :], 1, 0)
            + _dg(t1, W_r[1, :, :], 1, 0)
            + _dg(t2, W_r[2, :, :], 1, 0)
        )

    h0 = jax.nn.relu(cheb(xa, Wg0_r))
    h1 = jax.nn.relu(cheb(h0, Wg1_r))
    h2 = jax.nn.relu(cheb(h1, Wg2_r))
    h3 = jax.nn.relu(cheb(h2, Wg3_r))

    hg = h0.shape[1]
    Wc1 = Wc1_r[:]
    hc = jax.nn.relu(
        _dg(h0, Wc1[0:hg, :], 1, 0)
        + _dg(h1, Wc1[hg:2 * hg, :], 1, 0)
        + _dg(h2, Wc1[2 * hg:3 * hg, :], 1, 0)
        + _dg(h3, Wc1[3 * hg:, :], 1, 0)
        + bc1_r[:]
    )
    hc = hc / jnp.sqrt(1.0 + 1e-5) * gamma_r[:] + beta_r[:]
    logit = _dg(hc, Wc2_r[:], 1, 0) + bc2_r[:]
    m2 = jnp.max(logit, axis=1, keepdims=True)
    e2 = jnp.exp(logit - m2)
    out_o[:] = e2 / jnp.sum(e2, axis=1, keepdims=True)


def kernel(x1, x2, x3, Wq, bq, Wk, bk, Wv, bv, Wm1, bm1, Wm2, bm2,
           Wgc1, Wgc2, Wsgc, Wg0, Wg1, Wg2, Wg3, Wc1, bc1, gamma, beta,
           Wc2, bc2):
    n = x1.shape[0]
    din = x1.shape[1] + x2.shape[1]
    f32 = jnp.float32
    r2 = lambda v: v.reshape(1, -1)

    feats, tx1f, tx2f, tx1m, tx2m, ac, dinvc = pl.pallas_call(
        _adj_body,
        out_shape=(
            jax.ShapeDtypeStruct((n, din), f32),
            jax.ShapeDtypeStruct((n, din), f32),
            jax.ShapeDtypeStruct((n, din), f32),
            jax.ShapeDtypeStruct((n, din), f32),
            jax.ShapeDtypeStruct((n, din), f32),
            jax.ShapeDtypeStruct((n, n), f32),
            jax.ShapeDtypeStruct((n, 1), f32),
        ),
    )(x1, x2, x3, Wm1, r2(bm1), Wm2, r2(bm2))

    out = pl.pallas_call(
        _main_body,
        out_shape=jax.ShapeDtypeStruct((n, Wc2.shape[1]), f32),
    )(feats, tx1f, tx2f, tx1m, tx2m, ac, dinvc,
      Wgc1, Wgc2, Wsgc, Wq, r2(bq), Wk, r2(bk), Wv, r2(bv),
      Wg0, Wg1, Wg2, Wg3, Wc1, r2(bc1), r2(gamma), r2(beta), Wc2, r2(bc2))
    return out


# speedup vs baseline: 1.1360x; 1.1305x over previous
import jax
import jax.numpy as jnp
import numpy as np
from jax.experimental import pallas as pl


def _dg(a, b, ca, cb):
    return jax.lax.dot_general(
        a.astype(jnp.bfloat16), b.astype(jnp.bfloat16),
        (((ca,), (cb,)), ((), ())), preferred_element_type=jnp.float32
    )


def _dinv_of(A):
    deg = jnp.sum(A, axis=1, keepdims=True)
    pos = deg > 0
    return jnp.where(pos, 1.0 / jnp.sqrt(jnp.where(pos, deg, 1.0)), 0.0)


def _cheb_terms(A, dinv, x):
    t1 = -(dinv * _dg(A, dinv * x, 1, 0))
    t2 = 2.0 * (-(dinv * _dg(A, dinv * t1, 1, 0))) - x
    return t1, t2


def _body(x1_r, x2_r, x3_r, Wm1_r, bm1_r, Wm2_r, bm2_r,
          Wgc1_r, Wgc2_r, Wsgc_r, Wq_r, bq_r, Wk_r, bk_r, Wv_r, bv_r,
          Wg0_r, Wg1_r, Wg2_r, Wg3_r, Wc1_r, bc1_r, gamma_r, beta_r,
          Wc2_r, bc2_r, out_o):
    x1 = x1_r[:]
    x2 = x2_r[:]
    n = x1.shape[0]
    fusion = jnp.concatenate([x1, x2], axis=1)
    nrm = jnp.maximum(
        jnp.sqrt(jnp.sum(fusion * fusion, axis=1, keepdims=True)), 1e-12
    )
    xn = fusion / nrm
    sim = _dg(xn, xn, 1, 1)

    x3 = x3_r[:]
    pd = x3.shape[1]
    mu = jnp.mean(x3, axis=0, keepdims=True)
    sd = jnp.sqrt(jnp.mean((x3 - mu) * (x3 - mu), axis=0, keepdims=True))
    x3s = (x3 - mu) / sd
    Wm1 = Wm1_r[:]
    a = _dg(x3s, Wm1[:pd, :], 1, 0)
    bT = _dg(Wm1[pd:, :], x3s, 0, 1)
    bm1 = bm1_r[:]
    Wm2 = Wm2_r[:].astype(jnp.bfloat16).astype(jnp.float32)
    s = None
    for h in range(pd):
        t = jax.nn.relu(a[:, h:h + 1] + bT[h:h + 1, :] + bm1[0:1, h:h + 1])
        t = t.astype(jnp.bfloat16).astype(jnp.float32) * Wm2[h:h + 1, 0:1]
        s = t if s is None else s + t
    meta = jax.nn.sigmoid(s + bm2_r[0:1, 0:1])

    ii = jax.lax.broadcasted_iota(jnp.int32, (n, n), 0)
    jj = jax.lax.broadcasted_iota(jnp.int32, (n, n), 1)
    offd = ii != jj
    Af = jnp.where(offd & (sim > 0.8), sim, 0.0)
    Am = jnp.where(offd & (meta > 0.8), meta, 0.0)
    comb = sim + meta
    Ac = jnp.where(offd & (comb > 1.6), comb, 0.0)

    dinv_f = _dinv_of(Af)
    t1f, t2f = _cheb_terms(Af, dinv_f, fusion)
    dinv_m = _dinv_of(Am)
    t1m, t2m = _cheb_terms(Am, dinv_m, fusion)
    dinv_c = _dinv_of(Ac)

    din = fusion.shape[1]
    x1o = jax.nn.relu(
        _dg(fusion, Wgc1_r[0, :, :], 1, 0)
        + _dg(t1f, Wgc1_r[1, :, :], 1, 0)
        + _dg(t2f, Wgc1_r[2, :, :], 1, 0)
    )
    x2o = jax.nn.relu(
        _dg(fusion, Wgc2_r[0, :, :], 1, 0)
        + _dg(t1m, Wgc2_r[1, :, :], 1, 0)
        + _dg(t2m, Wgc2_r[2, :, :], 1, 0)
    )
    xm = (
        _dg(fusion, Wsgc_r[0, :, :], 1, 0)
        + _dg(t1f, Wsgc_r[1, :, :], 1, 0)
        + _dg(t2f, Wsgc_r[2, :, :], 1, 0)
    )

    def qkv(W_r, b_r):
        W = W_r[:]
        return (
            _dg(x1o, W[0:din, :], 1, 0)
            + _dg(x2o, W[din:2 * din, :], 1, 0)
            + _dg(xm, W[2 * din:, :], 1, 0)
            + b_r[:]
        )

    Q = qkv(Wq_r, bq_r)
    K = qkv(Wk_r, bk_r)
    V = qkv(Wv_r, bv_r)

    S = _dg(Q, K, 0, 0) * np.float32(1.0 / np.sqrt(float(din)))
    m = jnp.max(S, axis=0, keepdims=True)
    e = jnp.exp(S - m)
    att = e / jnp.sum(e, axis=0, keepdims=True)
    xa = _dg(V, att, 1, 0)

    def cheb(x, W_r):
        t1 = -(dinv_c * _dg(Ac, dinv_c * x, 1, 0))
        t2 = 2.0 * (-(dinv_c * _dg(Ac, dinv_c * t1, 1, 0))) - x
        return (
            _dg(x, W_r[0, :, :], 1, 0)
            + _dg(t1, W_r[1, :, :], 1, 0)
            + _dg(t2, W_r[2, :, :], 1, 0)
        )

    h0 = jax.nn.relu(cheb(xa, Wg0_r))
    h1 = jax.nn.relu(cheb(h0, Wg1_r))
    h2 = jax.nn.relu(cheb(h1, Wg2_r))
    h3 = jax.nn.relu(cheb(h2, Wg3_r))

    hg = h0.shape[1]
    Wc1 = Wc1_r[:]
    hc = jax.nn.relu(
        _dg(h0, Wc1[0:hg, :], 1, 0)
        + _dg(h1, Wc1[hg:2 * hg, :], 1, 0)
        + _dg(h2, Wc1[2 * hg:3 * hg, :], 1, 0)
        + _dg(h3, Wc1[3 * hg:, :], 1, 0)
        + bc1_r[:]
    )
    hc = hc / jnp.sqrt(1.0 + 1e-5) * gamma_r[:] + beta_r[:]
    logit = _dg(hc, Wc2_r[:], 1, 0) + bc2_r[:]
    m2 = jnp.max(logit, axis=1, keepdims=True)
    e2 = jnp.exp(logit - m2)
    out_o[:] = e2 / jnp.sum(e2, axis=1, keepdims=True)


def kernel(x1, x2, x3, Wq, bq, Wk, bk, Wv, bv, Wm1, bm1, Wm2, bm2,
           Wgc1, Wgc2, Wsgc, Wg0, Wg1, Wg2, Wg3, Wc1, bc1, gamma, beta,
           Wc2, bc2):
    n = x1.shape[0]
    r2 = lambda v: v.reshape(1, -1)
    return pl.pallas_call(
        _body,
        out_shape=jax.ShapeDtypeStruct((n, Wc2.shape[1]), jnp.float32),
    )(x1, x2, x3, Wm1, r2(bm1), Wm2, r2(bm2),
      Wgc1, Wgc2, Wsgc, Wq, r2(bq), Wk, r2(bk), Wv, r2(bv),
      Wg0, Wg1, Wg2, Wg3, Wc1, r2(bc1), r2(gamma), r2(beta), Wc2, r2(bc2))
